# baseline (speedup 1.0000x reference)
import functools

import jax
import jax.numpy as jnp
from jax import lax
from jax.experimental import pallas as pl
from jax.experimental.pallas import tpu as pltpu
from jax.experimental.pallas import tpu_sc as plsc

FM_H = 496
FM_W = 432
HW = FM_H * FM_W
B = 4
C = 64
P = 12000

NSEG = 8
SEG_COLS = FM_W // NSEG
SEG_CELLS = SEG_COLS * FM_H
PBLK = 2000
NBLK = P // PBLK
BLK_CHUNKS = PBLK // 16

CQ = 4
STRIPE_COLS = 8
NSTRIPE = FM_W // STRIPE_COLS
STRIPE_CELLS = STRIPE_COLS * FM_H
STRIPE_VECS = STRIPE_CELLS // 16
STRIPE_COLS_LOG2 = 3


def _fused_body(
    flag_hbm, xi_hbm, yi_hbm, x_hbm, out_hbm, pmap_hbm,
    flagv, xiv, yiv, mapband, xv, mapv, outv,
    sx, sm0, sm1, so0, so1,
):
    cid = lax.axis_index("c")
    sid = lax.axis_index("s")
    sm = (sm0, sm1)
    so = (so0, so1)
    zero16 = jnp.full((16,), 0, jnp.int32)
    zf16 = jnp.full((16,), 0.0, jnp.float32)

    def unit_geom(u):
        g = sid * 2 + u
        bb = g // 16
        c0 = (g % 16) * CQ
        b = 2 * cid + bb
        return b, bb, c0

    b0, _, c0_0 = unit_geom(0)
    xoff0 = pl.multiple_of((b0 * C + c0_0) * P, 8)
    xcp0 = pltpu.async_copy(x_hbm.at[pl.ds(xoff0, CQ * P)], xv, sx)

    bA_local = sid // NSEG
    bA = 2 * cid + bA_local
    seg = sid % NSEG
    x0 = seg * SEG_COLS

    neg1 = jnp.full((16,), -1, jnp.int32)

    @plsc.parallel_loop(0, SEG_CELLS // 16, 1, unroll=2)
    def _(i):
        mapband[pl.ds(i * 16, 16)] = neg1

    lanes = lax.iota(jnp.int32, 16)
    lane_masks = [lanes == jnp.full((16,), l, jnp.int32) for l in range(16)]

    def blk_body(blk, _):
        poff = pl.multiple_of(bA * P + blk * PBLK, 8)
        pltpu.sync_copy(flag_hbm.at[pl.ds(poff, PBLK)], flagv)
        pltpu.sync_copy(xi_hbm.at[pl.ds(poff, PBLK)], xiv)
        pltpu.sync_copy(yi_hbm.at[pl.ds(poff, PBLK)], yiv)

        def chunk_body(k, _):
            fl = flagv[pl.ds(k * 16, 16)]
            xc = xiv[pl.ds(k * 16, 16)]
            yc = yiv[pl.ds(k * 16, 16)]
            xl = xc - jnp.full((16,), 1, jnp.int32) * x0
            m = (fl == jnp.full((16,), 1, jnp.int32)) \
                & (xl >= jnp.full((16,), 0, jnp.int32)) \
                & (xl < jnp.full((16,), SEG_COLS, jnp.int32))
            cell = xl * jnp.full((16,), FM_H, jnp.int32) + yc
            cell = jnp.where(m, cell, jnp.full((16,), 0, jnp.int32))
            pvec = lanes + jnp.full((16,), 16, jnp.int32) * (blk * BLK_CHUNKS + k)
            for lm in lane_masks:
                plsc.store_scatter(mapband, [cell], pvec, mask=m & lm)
            return 0

        lax.fori_loop(0, BLK_CHUNKS, chunk_body, 0)
        return 0

    lax.fori_loop(0, NBLK, blk_body, 0)

    moff = pl.multiple_of(bA * HW + x0 * FM_H, 8)
    pltpu.sync_copy(mapband, pmap_hbm.at[pl.ds(moff, SEG_CELLS)])

    plsc.subcore_barrier()

    def map_slice(b, s):
        soff = pl.multiple_of(b * HW + s * STRIPE_CELLS, 8)
        return pmap_hbm.at[pl.ds(soff, STRIPE_CELLS)]

    def out_slice(b, c0, c, s):
        row0 = pl.multiple_of((b * C + c0 + c) * FM_W + s * STRIPE_COLS, 8)
        return out_hbm.at[pl.ds(row0, STRIPE_COLS), :]

    for u in range(2):
        b, bb, c0 = unit_geom(u)

        if u == 0:
            xcp0.wait()
        else:
            xoff = pl.multiple_of((b * C + c0) * P, 8)
            pltpu.sync_copy(x_hbm.at[pl.ds(xoff, CQ * P)], xv)

        for par in range(2):
            pltpu.async_copy(map_slice(b, par), mapv.at[par], sm[par])

        def sp_body(sp, _):
            for par in range(2):
                s = 2 * sp + par
                pltpu.make_async_copy(
                    map_slice(b, s), mapv.at[par], sm[par]
                ).wait()

                @pl.when(sp > 0)
                def _():
                    for c in range(CQ):
                        pltpu.make_async_copy(
                            outv.at[par, c],
                            out_slice(b, c0, c, s),
                            so[par],
                        ).wait()

                @plsc.parallel_loop(0, STRIPE_VECS, 1, unroll=8)
                def _(v):
                    w = lax.bitwise_and(v, STRIPE_COLS - 1)
                    h0 = lax.shift_left(
                        lax.shift_right_logical(v, STRIPE_COLS_LOG2), 4
                    )
                    m16 = mapv[par, pl.ds(w * FM_H + h0, 16)]
                    msk = m16 >= zero16
                    idx = jnp.maximum(m16, zero16)
                    for c in range(CQ):
                        gvals = plsc.load_gather(
                            xv, [jnp.full((16,), c * P, jnp.int32) + idx]
                        )
                        outv[par, c, w, pl.ds(h0, 16)] = jnp.where(
                            msk, gvals, zf16
                        )

                for c in range(CQ):
                    pltpu.async_copy(
                        outv.at[par, c],
                        out_slice(b, c0, c, s),
                        so[par],
                    )

                @pl.when(s + 2 < NSTRIPE)
                def _():
                    pltpu.async_copy(
                        map_slice(b, s + 2), mapv.at[par], sm[par]
                    )

            return 0

        lax.fori_loop(0, NSTRIPE // 2, sp_body, 0)

        for par in range(2):
            s_last = NSTRIPE - 2 + par
            for c in range(CQ):
                pltpu.make_async_copy(
                    outv.at[par, c],
                    out_slice(b, c0, c, s_last),
                    so[par],
                ).wait()


@functools.lru_cache(maxsize=1)
def _kernels():
    mesh = plsc.VectorSubcoreMesh(
        core_axis_name="c", subcore_axis_name="s", num_cores=2, num_subcores=16
    )
    params = pltpu.CompilerParams(needs_layout_passes=False)
    fused = pl.kernel(
        _fused_body,
        out_type=(
            jax.ShapeDtypeStruct((B * C * FM_W, FM_H), jnp.float32),
            jax.ShapeDtypeStruct((B * HW,), jnp.int32),
        ),
        mesh=mesh,
        compiler_params=params,
        scratch_types=[
            pltpu.VMEM((PBLK,), jnp.int32),
            pltpu.VMEM((PBLK,), jnp.int32),
            pltpu.VMEM((PBLK,), jnp.int32),
            pltpu.VMEM((SEG_CELLS,), jnp.int32),
            pltpu.VMEM((CQ * P,), jnp.float32),
            pltpu.VMEM((2, STRIPE_CELLS), jnp.int32),
            pltpu.VMEM((2, CQ, STRIPE_COLS, FM_H), jnp.float32),
            pltpu.SemaphoreType.DMA,
            pltpu.SemaphoreType.DMA,
            pltpu.SemaphoreType.DMA,
            pltpu.SemaphoreType.DMA,
            pltpu.SemaphoreType.DMA,
        ],
    )
    return fused


def kernel(x, inds):
    fused = _kernels()
    flag = inds[..., 0].astype(jnp.int32).reshape(-1)
    xi = inds[..., 1].astype(jnp.int32).reshape(-1)
    yi = inds[..., 2].astype(jnp.int32).reshape(-1)
    out, _ = fused(flag, xi, yi, x.reshape(-1))
    return out.reshape(B, C, FM_W, FM_H).transpose(0, 1, 3, 2)
